# Initial kernel scaffold; baseline (speedup 1.0000x reference)
import jax
import jax.numpy as jnp
from jax.experimental import pallas as pl


def kernel(x, edge_index, fc1_w, fc1_b, conv1_w, conv1_b, conv2_w, conv2_b):
    raise NotImplementedError("write your pallas kernel here")


# speedup vs baseline: 12.8516x; 12.8516x over previous
import functools

import jax
import jax.numpy as jnp
from jax import lax
from jax.experimental import pallas as pl
from jax.experimental.pallas import tpu as pltpu
from jax.experimental.pallas import tpu_sc as plsc

N = 10000
E = 320000
D_IN = 128
D_H = 128
D_OUT = 64

NC = 2
NS = 16
NW = NC * NS
CHUNK = 128
K = -(-E // (NW * CHUNK))
EPAD = NW * K * CHUNK
SINK = N
ROWS_PER_TILE = 640
NSINK = NS * ROWS_PER_TILE

_mesh = plsc.VectorSubcoreMesh(core_axis_name="c", subcore_axis_name="s")


@functools.partial(
    pl.kernel,
    mesh=_mesh,
    out_type=jax.ShapeDtypeStruct((NC, NSINK), jnp.float32),
    scratch_types=[
        pltpu.VMEM((K, CHUNK), jnp.int32),
        pltpu.VMEM((CHUNK,), jnp.float32),
        pltpu.VMEM((ROWS_PER_TILE,), jnp.float32),
        pltpu.VMEM_SHARED((NSINK,), jnp.float32),
    ],
)
def _degree_kernel(dst_hbm, out_hbm, dst_v, ones_v, zero_v, acc):
    cid = lax.axis_index("c")
    sid = lax.axis_index("s")
    wid = cid * NS + sid
    pltpu.sync_copy(dst_hbm.at[wid], dst_v)

    for i in range(CHUNK // 16):
        ones_v[pl.ds(i * 16, 16)] = jnp.ones((16,), jnp.float32)

    def _z(i, carry):
        zero_v[pl.ds(i * 16, 16)] = jnp.zeros((16,), jnp.float32)
        return carry

    lax.fori_loop(0, ROWS_PER_TILE // 16, _z, 0)
    pltpu.sync_copy(zero_v, acc.at[pl.ds(sid * ROWS_PER_TILE, ROWS_PER_TILE)])
    plsc.subcore_barrier()

    def _scatter(j, carry):
        pltpu.sync_copy(ones_v, acc.at[dst_v.at[j]], add=True)
        return carry

    lax.fori_loop(0, K, _scatter, 0)
    plsc.subcore_barrier()
    pltpu.sync_copy(
        acc.at[pl.ds(sid * ROWS_PER_TILE, ROWS_PER_TILE)],
        out_hbm.at[cid, pl.ds(sid * ROWS_PER_TILE, ROWS_PER_TILE)],
    )


def _make_row_scatter(D):
    @functools.partial(
        pl.kernel,
        mesh=_mesh,
        out_type=jax.ShapeDtypeStruct((NC, NSINK, D), jnp.float32),
        scratch_types=[
            pltpu.VMEM((K, CHUNK), jnp.int32),
            pltpu.VMEM((K, CHUNK), jnp.int32),
            pltpu.VMEM((CHUNK, D), jnp.float32),
            pltpu.VMEM_SHARED((NSINK, D), jnp.float32),
            pltpu.SemaphoreType.DMA,
        ],
    )
    def _row_scatter(g_hbm, src_hbm, dst_hbm, out_hbm,
                     src_v, dst_v, rows_v, acc, sem):
        cid = lax.axis_index("c")
        sid = lax.axis_index("s")
        wid = cid * NS + sid
        pltpu.sync_copy(src_hbm.at[wid], src_v)
        pltpu.sync_copy(dst_hbm.at[wid], dst_v)

        def _z(i, carry):
            r = i // (D // 16)
            c = lax.rem(i, D // 16)
            rows_v[r, pl.ds(c * 16, 16)] = jnp.zeros((16,), jnp.float32)
            return carry

        lax.fori_loop(0, CHUNK * D // 16, _z, 0)
        for b in range(ROWS_PER_TILE // CHUNK):
            pltpu.sync_copy(
                rows_v,
                acc.at[pl.ds(sid * ROWS_PER_TILE + b * CHUNK, CHUNK)],
            )
        plsc.subcore_barrier()

        def _edge_chunk(j, carry):
            pltpu.async_copy(g_hbm.at[src_v.at[j]], rows_v, sem).wait()
            pltpu.sync_copy(rows_v, acc.at[dst_v.at[j]], add=True)
            return carry

        lax.fori_loop(0, K, _edge_chunk, 0)
        plsc.subcore_barrier()
        pltpu.sync_copy(
            acc.at[pl.ds(sid * ROWS_PER_TILE, ROWS_PER_TILE)],
            out_hbm.at[cid, pl.ds(sid * ROWS_PER_TILE, ROWS_PER_TILE)],
        )

    return _row_scatter


_row_scatter_h = _make_row_scatter(D_H)


def _dot(a, b):
    return jnp.dot(a, b, precision=lax.Precision.HIGHEST,
                   preferred_element_type=jnp.float32)


def _tc1_body(x_ref, w1_ref, b1_ref, wc1_ref, d0_ref, d1_ref, g1_ref):
    h = jnp.maximum(_dot(x_ref[...], w1_ref[...]) + b1_ref[...], 0.0)
    dis = lax.rsqrt(d0_ref[...] + d1_ref[...])
    g1_ref[...] = _dot(h, wc1_ref[...]) * dis


def _tc2_body(s0_ref, s1_ref, g1_ref, b1_ref, wc2_ref, d0_ref, d1_ref, g2_ref):
    dis = lax.rsqrt(d0_ref[...] + d1_ref[...])
    h = jnp.maximum(
        dis * (s0_ref[...] + s1_ref[...] + g1_ref[...]) + b1_ref[...], 0.0)
    g2_ref[...] = jnp.concatenate(
        [_dot(h, wc2_ref[...]) * dis, jnp.zeros((N, D_H - D_OUT), jnp.float32)],
        axis=1)


def _tc3_body(s0_ref, s1_ref, g2_ref, b2_ref, d0_ref, d1_ref, out_ref):
    dis = lax.rsqrt(d0_ref[...] + d1_ref[...])
    agg = (s0_ref[...] + s1_ref[...] + g2_ref[...])[:, :D_OUT]
    out_ref[...] = dis * agg + b2_ref[...]


_tc1 = pl.pallas_call(
    _tc1_body, out_shape=jax.ShapeDtypeStruct((N, D_H), jnp.float32))
_tc2 = pl.pallas_call(
    _tc2_body, out_shape=jax.ShapeDtypeStruct((N, D_H), jnp.float32))
_tc3 = pl.pallas_call(
    _tc3_body, out_shape=jax.ShapeDtypeStruct((N, D_OUT), jnp.float32))


def kernel(x, edge_index, fc1_w, fc1_b, conv1_w, conv1_b, conv2_w, conv2_b):
    src = edge_index[0]
    dst = edge_index[1]
    pad = EPAD - E
    src3 = jnp.concatenate(
        [src, jnp.zeros((pad,), jnp.int32)]).reshape(NW, K, CHUNK)
    dst3 = jnp.concatenate(
        [dst, jnp.full((pad,), SINK, jnp.int32)]).reshape(NW, K, CHUNK)

    degp = _degree_kernel(dst3)
    d0 = degp[0, :N, None] + 1.0
    d1 = degp[1, :N, None]

    g1 = _tc1(x, fc1_w, fc1_b.reshape(1, D_H), conv1_w, d0, d1)
    s1 = _row_scatter_h(g1, src3, dst3)
    g2 = _tc2(s1[0, :N], s1[1, :N], g1, conv1_b.reshape(1, D_H), conv2_w,
              d0, d1)
    s2 = _row_scatter_h(g2, src3, dst3)
    out = _tc3(s2[0, :N], s2[1, :N], g2, conv2_b.reshape(1, D_OUT), d0, d1)
    return out
